# baseline (speedup 1.0000x reference)
import jax
import jax.numpy as jnp
from jax import lax
from jax.experimental import pallas as pl
from jax.experimental.pallas import tpu as pltpu
from jax.experimental.pallas import tpu_sc as plsc

N = 100000
E = 3200000
NC = 2
NS = 16
NW = NC * NS
BLK_E = 768
NBLK = 132
EW = NBLK * BLK_E
E_PAD = NW * EW
ACC_R = 100096
ZR = ACC_R // NS


def _sc_aggregate_body(t_hbm, s_hbm, d_hbm, out_hbm,
                       sA, dA, rowsA, sB, dB, rowsB, acc, gsA, gsB, ssA, ssB):
    cid = lax.axis_index("c")
    sid = lax.axis_index("s")
    wid = sid * NC + cid

    def zrow(i, carry):
        rowsA[i] = jnp.zeros((16,), jnp.float32)
        return carry
    lax.fori_loop(0, BLK_E, zrow, 0)
    base = sid * ZR
    for k in range(ZR // BLK_E):
        pltpu.sync_copy(rowsA, acc.at[pl.ds(base + k * BLK_E, BLK_E)])
    rem = ZR % BLK_E
    if rem:
        pltpu.sync_copy(rowsA.at[pl.ds(0, rem)],
                        acc.at[pl.ds(base + (ZR // BLK_E) * BLK_E, rem)])
    plsc.subcore_barrier()

    ebase = wid * ((NBLK + 1) * BLK_E)

    def load_fire(b, sbuf, dbuf, rows, gs):
        pltpu.sync_copy(s_hbm.at[pl.ds(ebase + b * BLK_E, BLK_E)], sbuf)
        pltpu.sync_copy(d_hbm.at[pl.ds(ebase + b * BLK_E, BLK_E)], dbuf)
        pltpu.async_copy(t_hbm.at[sbuf], rows, gs)

    def wait_gathers(sbuf, rows, gs):
        pltpu.make_async_copy(t_hbm.at[sbuf], rows, gs).wait()

    def scatter_drain(dbuf, rows, ss):
        pltpu.async_copy(rows, acc.at[dbuf], ss, add=True)
        pltpu.make_async_copy(rows, acc.at[dbuf], ss).wait()

    load_fire(0, sA, dA, rowsA, gsA)

    def pair(p, carry):
        b0 = 2 * p
        load_fire(b0 + 1, sB, dB, rowsB, gsB)
        wait_gathers(sA, rowsA, gsA)
        scatter_drain(dA, rowsA, ssA)
        load_fire(b0 + 2, sA, dA, rowsA, gsA)
        wait_gathers(sB, rowsB, gsB)
        scatter_drain(dB, rowsB, ssB)
        return carry

    lax.fori_loop(0, NBLK // 2, pair, 0)
    wait_gathers(sA, rowsA, gsA)

    plsc.subcore_barrier()
    pltpu.sync_copy(acc.at[pl.ds(sid * ZR, ZR)],
                    out_hbm.at[cid].at[pl.ds(sid * ZR, ZR)])


def _sc_aggregate(table, src_hbm, dst_hbm):
    mesh = plsc.VectorSubcoreMesh(core_axis_name="c", subcore_axis_name="s")
    f = pl.kernel(
        _sc_aggregate_body,
        out_type=jax.ShapeDtypeStruct((NC, ACC_R, 16), jnp.float32),
        mesh=mesh,
        compiler_params=pltpu.CompilerParams(use_tc_tiling_on_sc=False),
        scratch_types=[
            pltpu.VMEM((BLK_E,), jnp.int32),
            pltpu.VMEM((BLK_E,), jnp.int32),
            pltpu.VMEM((BLK_E, 16), jnp.float32),
            pltpu.VMEM((BLK_E,), jnp.int32),
            pltpu.VMEM((BLK_E,), jnp.int32),
            pltpu.VMEM((BLK_E, 16), jnp.float32),
            pltpu.VMEM_SHARED((ACC_R, 16), jnp.float32),
            pltpu.SemaphoreType.DMA,
            pltpu.SemaphoreType.DMA,
            pltpu.SemaphoreType.DMA,
            pltpu.SemaphoreType.DMA,
        ],
    )
    return f(table, src_hbm, dst_hbm)


ROWS128 = ACC_R // 8
_BN = ROWS128 // 4


def _tc_layer1_body(p_ref, t_ref, s1_ref, wl_ref, wr_ref, b_ref, h_ref):
    s = p_ref[0] + p_ref[1]
    cnt = jnp.maximum(
        jnp.dot(s, s1_ref[...], preferred_element_type=jnp.float32), 1.0)
    mean = s / cnt
    h = (jnp.dot(mean, wl_ref[...], preferred_element_type=jnp.float32)
         + jnp.dot(t_ref[...], wr_ref[...], preferred_element_type=jnp.float32)
         + b_ref[...])
    h_ref[...] = jnp.maximum(h, 0.0)


def _tc_layer2_body(p2_ref, p1_ref, h_ref, s1_ref, wl_ref, wr_ref, b_ref, o_ref):
    s1 = p1_ref[0] + p1_ref[1]
    cnt = jnp.maximum(
        jnp.dot(s1, s1_ref[...], preferred_element_type=jnp.float32), 1.0)
    s2 = p2_ref[0] + p2_ref[1]
    o_ref[...] = (jnp.dot(s2 / cnt, wl_ref[...], preferred_element_type=jnp.float32)
                  + jnp.dot(h_ref[...], wr_ref[...], preferred_element_type=jnp.float32)
                  + b_ref[...])


def _tc_layer1(p1, table1, S1, WLbd, WRbd, b1t):
    return pl.pallas_call(
        _tc_layer1_body,
        grid=(ROWS128 // _BN,),
        in_specs=[
            pl.BlockSpec((NC, _BN, 128), lambda i: (0, i, 0)),
            pl.BlockSpec((_BN, 128), lambda i: (i, 0)),
            pl.BlockSpec((128, 128), lambda i: (0, 0)),
            pl.BlockSpec((128, 128), lambda i: (0, 0)),
            pl.BlockSpec((128, 128), lambda i: (0, 0)),
            pl.BlockSpec((1, 128), lambda i: (0, 0)),
        ],
        out_specs=pl.BlockSpec((_BN, 128), lambda i: (i, 0)),
        out_shape=jax.ShapeDtypeStruct((ROWS128, 128), jnp.float32),
    )(p1, table1, S1, WLbd, WRbd, b1t)


def _tc_layer2(p2, p1, h, S1, WLbd, WRbd, b2t):
    return pl.pallas_call(
        _tc_layer2_body,
        grid=(ROWS128 // _BN,),
        in_specs=[
            pl.BlockSpec((NC, _BN, 128), lambda i: (0, i, 0)),
            pl.BlockSpec((NC, _BN, 128), lambda i: (0, i, 0)),
            pl.BlockSpec((_BN, 128), lambda i: (i, 0)),
            pl.BlockSpec((128, 128), lambda i: (0, 0)),
            pl.BlockSpec((128, 128), lambda i: (0, 0)),
            pl.BlockSpec((128, 128), lambda i: (0, 0)),
            pl.BlockSpec((1, 128), lambda i: (0, 0)),
        ],
        out_specs=pl.BlockSpec((_BN, 128), lambda i: (i, 0)),
        out_shape=jax.ShapeDtypeStruct((ROWS128, 128), jnp.float32),
    )(p2, p1, h, S1, WLbd, WRbd, b2t)


def kernel(x, edge_index, W1_l, b1, W1_r, W2_l, b2, W2_r):
    src = edge_index[0]
    dst = edge_index[1]
    pad = E_PAD - E
    src_p = jnp.concatenate([src, jnp.zeros((pad,), jnp.int32)])
    dst_p = jnp.concatenate([dst, jnp.full((pad,), N, jnp.int32)])
    s3 = jnp.pad(src_p.reshape(NW, NBLK, BLK_E), ((0, 0), (0, 1), (0, 0)))
    d3 = jnp.pad(dst_p.reshape(NW, NBLK, BLK_E), ((0, 0), (0, 1), (0, 0)))
    src_flat = s3.reshape(-1)
    dst_flat = d3.reshape(-1)

    xr = x.reshape(N // 8, 8, 8)
    t1g = jnp.concatenate(
        [xr, jnp.ones((N // 8, 8, 1), jnp.float32),
         jnp.zeros((N // 8, 8, 7), jnp.float32)], axis=2)
    table1_128 = jnp.pad(t1g.reshape(N // 8, 128),
                         ((0, ROWS128 - N // 8), (0, 0)))

    eye8 = jnp.eye(8, dtype=jnp.float32)
    W1l_pad = jnp.concatenate([W1_l, jnp.zeros((8, 16), jnp.float32)], axis=0)
    W1r_pad = jnp.concatenate([W1_r, jnp.zeros((8, 16), jnp.float32)], axis=0)
    S16 = jnp.zeros((16, 16), jnp.float32).at[8, :].set(1.0)
    S1 = jnp.kron(eye8, S16)
    WL1 = jnp.kron(eye8, W1l_pad)
    WR1 = jnp.kron(eye8, W1r_pad)
    WL2 = jnp.kron(eye8, W2_l)
    WR2 = jnp.kron(eye8, W2_r)
    b1t = jnp.tile(b1, 8).reshape(1, 128)
    b2t = jnp.tile(b2, 8).reshape(1, 128)

    p1 = _sc_aggregate(table1_128.reshape(ACC_R, 16), src_flat, dst_flat)
    p1_128 = p1.reshape(NC, ROWS128, 128)
    h128 = _tc_layer1(p1_128, table1_128, S1, WL1, WR1, b1t)
    p2 = _sc_aggregate(h128.reshape(ACC_R, 16), src_flat, dst_flat)
    p2_128 = p2.reshape(NC, ROWS128, 128)
    out128 = _tc_layer2(p2_128, p1_128, h128, S1, WL2, WR2, b2t)
    return out128.reshape(ACC_R, 16)[:N]


# speedup vs baseline: 1.3701x; 1.3701x over previous
import jax
import jax.numpy as jnp
from jax import lax
from jax.experimental import pallas as pl
from jax.experimental.pallas import tpu as pltpu
from jax.experimental.pallas import tpu_sc as plsc

N = 100000
E = 3200000
NC = 2
NS = 16
NW = NC * NS
BLK_E = 512
NBLK = 196
EW = NBLK * BLK_E
E_PAD = NW * EW
ACC_R = 100096
ZR = ACC_R // NS


def _sc_aggregate_body(t_hbm, s_hbm, d_hbm, out_hbm,
                       sA, dA, rowsA, sB, dB, rowsB, acc,
                       gsA, gsB, ssA, ssB, isA, isB):
    cid = lax.axis_index("c")
    sid = lax.axis_index("s")
    wid = sid * NC + cid

    def zrow(i, carry):
        rowsA[i] = jnp.zeros((16,), jnp.float32)
        return carry
    lax.fori_loop(0, BLK_E, zrow, 0)
    base = sid * ZR
    for k in range(ZR // BLK_E):
        pltpu.sync_copy(rowsA, acc.at[pl.ds(base + k * BLK_E, BLK_E)])
    rem = ZR % BLK_E
    if rem:
        pltpu.sync_copy(rowsA.at[pl.ds(0, rem)],
                        acc.at[pl.ds(base + (ZR // BLK_E) * BLK_E, rem)])
    plsc.subcore_barrier()

    ebase = wid * ((NBLK + 2) * BLK_E)

    def fire_idx(b, sbuf, dbuf, isem):
        pltpu.async_copy(s_hbm.at[pl.ds(ebase + b * BLK_E, BLK_E)], sbuf, isem)
        pltpu.async_copy(d_hbm.at[pl.ds(ebase + b * BLK_E, BLK_E)], dbuf, isem)

    def wait_idx(b, sbuf, dbuf, isem):
        pltpu.make_async_copy(s_hbm.at[pl.ds(ebase + b * BLK_E, BLK_E)], sbuf, isem).wait()
        pltpu.make_async_copy(d_hbm.at[pl.ds(ebase + b * BLK_E, BLK_E)], dbuf, isem).wait()

    def fire_g(sbuf, rows, gs):
        pltpu.async_copy(t_hbm.at[sbuf], rows, gs)

    def wait_gathers(sbuf, rows, gs):
        pltpu.make_async_copy(t_hbm.at[sbuf], rows, gs).wait()

    def scatter_drain(dbuf, rows, ss):
        pltpu.async_copy(rows, acc.at[dbuf], ss, add=True)
        pltpu.make_async_copy(rows, acc.at[dbuf], ss).wait()

    fire_idx(0, sA, dA, isA)
    wait_idx(0, sA, dA, isA)
    fire_g(sA, rowsA, gsA)
    fire_idx(1, sB, dB, isB)

    def pair(p, carry):
        b0 = 2 * p
        wait_idx(b0 + 1, sB, dB, isB)
        fire_g(sB, rowsB, gsB)
        wait_gathers(sA, rowsA, gsA)
        scatter_drain(dA, rowsA, ssA)
        fire_idx(b0 + 2, sA, dA, isA)
        wait_gathers(sB, rowsB, gsB)
        scatter_drain(dB, rowsB, ssB)
        fire_idx(b0 + 3, sB, dB, isB)
        wait_idx(b0 + 2, sA, dA, isA)
        fire_g(sA, rowsA, gsA)
        return carry

    lax.fori_loop(0, NBLK // 2, pair, 0)
    wait_gathers(sA, rowsA, gsA)
    wait_idx(NBLK + 1, sB, dB, isB)

    plsc.subcore_barrier()
    pltpu.sync_copy(acc.at[pl.ds(sid * ZR, ZR)],
                    out_hbm.at[cid].at[pl.ds(sid * ZR, ZR)])


def _sc_aggregate(table, src_hbm, dst_hbm):
    mesh = plsc.VectorSubcoreMesh(core_axis_name="c", subcore_axis_name="s")
    f = pl.kernel(
        _sc_aggregate_body,
        out_type=jax.ShapeDtypeStruct((NC, ACC_R, 16), jnp.float32),
        mesh=mesh,
        compiler_params=pltpu.CompilerParams(use_tc_tiling_on_sc=False),
        scratch_types=[
            pltpu.VMEM((BLK_E,), jnp.int32),
            pltpu.VMEM((BLK_E,), jnp.int32),
            pltpu.VMEM((BLK_E, 16), jnp.float32),
            pltpu.VMEM((BLK_E,), jnp.int32),
            pltpu.VMEM((BLK_E,), jnp.int32),
            pltpu.VMEM((BLK_E, 16), jnp.float32),
            pltpu.VMEM_SHARED((ACC_R, 16), jnp.float32),
            pltpu.SemaphoreType.DMA,
            pltpu.SemaphoreType.DMA,
            pltpu.SemaphoreType.DMA,
            pltpu.SemaphoreType.DMA,
            pltpu.SemaphoreType.DMA,
            pltpu.SemaphoreType.DMA,
        ],
    )
    return f(table, src_hbm, dst_hbm)


ROWS128 = ACC_R // 8
_BN = ROWS128 // 4


def _tc_layer1_body(p_ref, t_ref, s1_ref, wl_ref, wr_ref, b_ref, h_ref):
    s = p_ref[0] + p_ref[1]
    cnt = jnp.maximum(
        jnp.dot(s, s1_ref[...], preferred_element_type=jnp.float32), 1.0)
    mean = s / cnt
    h = (jnp.dot(mean, wl_ref[...], preferred_element_type=jnp.float32)
         + jnp.dot(t_ref[...], wr_ref[...], preferred_element_type=jnp.float32)
         + b_ref[...])
    h_ref[...] = jnp.maximum(h, 0.0)


def _tc_layer2_body(p2_ref, p1_ref, h_ref, s1_ref, wl_ref, wr_ref, b_ref, o_ref):
    s1 = p1_ref[0] + p1_ref[1]
    cnt = jnp.maximum(
        jnp.dot(s1, s1_ref[...], preferred_element_type=jnp.float32), 1.0)
    s2 = p2_ref[0] + p2_ref[1]
    o_ref[...] = (jnp.dot(s2 / cnt, wl_ref[...], preferred_element_type=jnp.float32)
                  + jnp.dot(h_ref[...], wr_ref[...], preferred_element_type=jnp.float32)
                  + b_ref[...])


def _tc_layer1(p1, table1, S1, WLbd, WRbd, b1t):
    return pl.pallas_call(
        _tc_layer1_body,
        grid=(ROWS128 // _BN,),
        in_specs=[
            pl.BlockSpec((NC, _BN, 128), lambda i: (0, i, 0)),
            pl.BlockSpec((_BN, 128), lambda i: (i, 0)),
            pl.BlockSpec((128, 128), lambda i: (0, 0)),
            pl.BlockSpec((128, 128), lambda i: (0, 0)),
            pl.BlockSpec((128, 128), lambda i: (0, 0)),
            pl.BlockSpec((1, 128), lambda i: (0, 0)),
        ],
        out_specs=pl.BlockSpec((_BN, 128), lambda i: (i, 0)),
        out_shape=jax.ShapeDtypeStruct((ROWS128, 128), jnp.float32),
    )(p1, table1, S1, WLbd, WRbd, b1t)


def _tc_layer2(p2, p1, h, S1, WLbd, WRbd, b2t):
    return pl.pallas_call(
        _tc_layer2_body,
        grid=(ROWS128 // _BN,),
        in_specs=[
            pl.BlockSpec((NC, _BN, 128), lambda i: (0, i, 0)),
            pl.BlockSpec((NC, _BN, 128), lambda i: (0, i, 0)),
            pl.BlockSpec((_BN, 128), lambda i: (i, 0)),
            pl.BlockSpec((128, 128), lambda i: (0, 0)),
            pl.BlockSpec((128, 128), lambda i: (0, 0)),
            pl.BlockSpec((128, 128), lambda i: (0, 0)),
            pl.BlockSpec((1, 128), lambda i: (0, 0)),
        ],
        out_specs=pl.BlockSpec((_BN, 128), lambda i: (i, 0)),
        out_shape=jax.ShapeDtypeStruct((ROWS128, 128), jnp.float32),
    )(p2, p1, h, S1, WLbd, WRbd, b2t)


def kernel(x, edge_index, W1_l, b1, W1_r, W2_l, b2, W2_r):
    src = edge_index[0]
    dst = edge_index[1]
    pad = E_PAD - E
    src_p = jnp.concatenate([src, jnp.zeros((pad,), jnp.int32)])
    dst_p = jnp.concatenate([dst, jnp.full((pad,), N, jnp.int32)])
    s3 = jnp.pad(src_p.reshape(NW, NBLK, BLK_E), ((0, 0), (0, 2), (0, 0)))
    d3 = jnp.pad(dst_p.reshape(NW, NBLK, BLK_E), ((0, 0), (0, 2), (0, 0)))
    src_flat = s3.reshape(-1)
    dst_flat = d3.reshape(-1)

    xr = x.reshape(N // 8, 8, 8)
    t1g = jnp.concatenate(
        [xr, jnp.ones((N // 8, 8, 1), jnp.float32),
         jnp.zeros((N // 8, 8, 7), jnp.float32)], axis=2)
    table1_128 = jnp.pad(t1g.reshape(N // 8, 128),
                         ((0, ROWS128 - N // 8), (0, 0)))

    eye8 = jnp.eye(8, dtype=jnp.float32)
    W1l_pad = jnp.concatenate([W1_l, jnp.zeros((8, 16), jnp.float32)], axis=0)
    W1r_pad = jnp.concatenate([W1_r, jnp.zeros((8, 16), jnp.float32)], axis=0)
    S16 = jnp.zeros((16, 16), jnp.float32).at[8, :].set(1.0)
    S1 = jnp.kron(eye8, S16)
    WL1 = jnp.kron(eye8, W1l_pad)
    WR1 = jnp.kron(eye8, W1r_pad)
    WL2 = jnp.kron(eye8, W2_l)
    WR2 = jnp.kron(eye8, W2_r)
    b1t = jnp.tile(b1, 8).reshape(1, 128)
    b2t = jnp.tile(b2, 8).reshape(1, 128)

    p1 = _sc_aggregate(table1_128.reshape(ACC_R, 16), src_flat, dst_flat)
    p1_128 = p1.reshape(NC, ROWS128, 128)
    h128 = _tc_layer1(p1_128, table1_128, S1, WL1, WR1, b1t)
    p2 = _sc_aggregate(h128.reshape(ACC_R, 16), src_flat, dst_flat)
    p2_128 = p2.reshape(NC, ROWS128, 128)
    out128 = _tc_layer2(p2_128, p1_128, h128, S1, WL2, WR2, b2t)
    return out128.reshape(ACC_R, 16)[:N]
